# baseline (device time: 15449 ns/iter reference)
import jax
import jax.numpy as jnp
from jax import lax
from jax.experimental import pallas as pl
from jax.experimental.pallas import tpu as pltpu

N_DEV = 32
N_GLOBAL = 16384
EPS = 1e-5


def kernel(x, gamma):
    m, n_loc = x.shape
    gamma2d = gamma.reshape(1, n_loc)

    def body(x_ref, g_ref, out_ref, comm_ref, send_sems, recv_sems):
        my_pos = lax.axis_index("i")

        xv = x_ref[...]
        partial = jnp.sum(xv * xv, axis=1).reshape(1, m)
        comm_ref[my_pos] = partial

        barrier = pltpu.get_barrier_semaphore()
        for p in range(N_DEV):
            @pl.when(p != my_pos)
            def _(p=p):
                pl.semaphore_signal(
                    barrier, inc=1,
                    device_id=(p,), device_id_type=pl.DeviceIdType.MESH,
                )
        pl.semaphore_wait(barrier, N_DEV - 1)

        for p in range(N_DEV):
            @pl.when(p != my_pos)
            def _(p=p):
                rdma = pltpu.make_async_remote_copy(
                    src_ref=comm_ref.at[my_pos],
                    dst_ref=comm_ref.at[my_pos],
                    send_sem=send_sems.at[p],
                    recv_sem=recv_sems.at[my_pos],
                    device_id=(p,),
                    device_id_type=pl.DeviceIdType.MESH,
                )
                rdma.start()

        for p in range(N_DEV):
            @pl.when(p != my_pos)
            def _(p=p):
                rdma = pltpu.make_async_remote_copy(
                    src_ref=comm_ref.at[my_pos],
                    dst_ref=comm_ref.at[p],
                    send_sem=send_sems.at[p],
                    recv_sem=recv_sems.at[p],
                    device_id=(p,),
                    device_id_type=pl.DeviceIdType.MESH,
                )
                rdma.wait_recv()

        for p in range(N_DEV):
            @pl.when(p != my_pos)
            def _(p=p):
                rdma = pltpu.make_async_remote_copy(
                    src_ref=comm_ref.at[my_pos],
                    dst_ref=comm_ref.at[my_pos],
                    send_sem=send_sems.at[p],
                    recv_sem=recv_sems.at[my_pos],
                    device_id=(p,),
                    device_id_type=pl.DeviceIdType.MESH,
                )
                rdma.wait_send()

        total = jnp.sum(comm_ref[...], axis=0)
        inv = lax.rsqrt(total / N_GLOBAL + EPS)
        inv_col = inv.reshape(m, 1)
        out_ref[...] = xv * g_ref[...] * inv_col

    return pl.pallas_call(
        body,
        out_shape=jax.ShapeDtypeStruct((m, n_loc), jnp.float32),
        in_specs=[
            pl.BlockSpec(memory_space=pltpu.VMEM),
            pl.BlockSpec(memory_space=pltpu.VMEM),
        ],
        out_specs=pl.BlockSpec(memory_space=pltpu.VMEM),
        scratch_shapes=[
            pltpu.VMEM((N_DEV, 1, m), jnp.float32),
            pltpu.SemaphoreType.DMA((N_DEV,)),
            pltpu.SemaphoreType.DMA((N_DEV,)),
        ],
        compiler_params=pltpu.CompilerParams(collective_id=0),
    )(x, gamma2d)


# device time: 15052 ns/iter; 1.0264x vs baseline; 1.0264x over previous
import jax
import jax.numpy as jnp
from jax import lax
from jax.experimental import pallas as pl
from jax.experimental.pallas import tpu as pltpu

N_DEV = 32
N_GLOBAL = 16384
EPS = 1e-5


def kernel(x, gamma):
    m, n_loc = x.shape
    gamma2d = gamma.reshape(1, n_loc)

    def body(x_ref, g_ref, out_ref, comm_ref, send_sems, recv_sems):
        my_pos = lax.axis_index("i")

        barrier = pltpu.get_barrier_semaphore()
        for p in range(N_DEV):
            @pl.when(p != my_pos)
            def _(p=p):
                pl.semaphore_signal(
                    barrier, inc=1,
                    device_id=(p,), device_id_type=pl.DeviceIdType.MESH,
                )

        xv = x_ref[...]
        partial = jnp.sum(xv * xv, axis=1).reshape(6, 128)
        comm_ref[my_pos] = partial

        pl.semaphore_wait(barrier, N_DEV - 1)

        for p in range(N_DEV):
            @pl.when(p != my_pos)
            def _(p=p):
                rdma = pltpu.make_async_remote_copy(
                    src_ref=comm_ref.at[my_pos],
                    dst_ref=comm_ref.at[my_pos],
                    send_sem=send_sems.at[p],
                    recv_sem=recv_sems.at[my_pos],
                    device_id=(p,),
                    device_id_type=pl.DeviceIdType.MESH,
                )
                rdma.start()

        scaled = xv * g_ref[...]

        for p in range(N_DEV):
            @pl.when(p != my_pos)
            def _(p=p):
                rdma = pltpu.make_async_remote_copy(
                    src_ref=comm_ref.at[my_pos],
                    dst_ref=comm_ref.at[p],
                    send_sem=send_sems.at[p],
                    recv_sem=recv_sems.at[p],
                    device_id=(p,),
                    device_id_type=pl.DeviceIdType.MESH,
                )
                rdma.wait_recv()

        for p in range(N_DEV):
            @pl.when(p != my_pos)
            def _(p=p):
                rdma = pltpu.make_async_remote_copy(
                    src_ref=comm_ref.at[my_pos],
                    dst_ref=comm_ref.at[my_pos],
                    send_sem=send_sems.at[p],
                    recv_sem=recv_sems.at[my_pos],
                    device_id=(p,),
                    device_id_type=pl.DeviceIdType.MESH,
                )
                rdma.wait_send()

        total = jnp.sum(comm_ref[...], axis=0)
        inv = lax.rsqrt(total / N_GLOBAL + EPS)
        inv_col = jnp.swapaxes(inv.reshape(1, m), 0, 1)
        out_ref[...] = scaled * inv_col

    return pl.pallas_call(
        body,
        out_shape=jax.ShapeDtypeStruct((m, n_loc), jnp.float32),
        in_specs=[
            pl.BlockSpec(memory_space=pltpu.VMEM),
            pl.BlockSpec(memory_space=pltpu.VMEM),
        ],
        out_specs=pl.BlockSpec(memory_space=pltpu.VMEM),
        scratch_shapes=[
            pltpu.VMEM((N_DEV, 6, 128), jnp.float32),
            pltpu.SemaphoreType.DMA((N_DEV,)),
            pltpu.SemaphoreType.DMA((N_DEV,)),
        ],
        compiler_params=pltpu.CompilerParams(collective_id=0),
    )(x, gamma2d)
